# Initial kernel scaffold; baseline (speedup 1.0000x reference)
import jax
import jax.numpy as jnp
from jax.experimental import pallas as pl


def kernel(x, position_matrix):
    raise NotImplementedError("write your pallas kernel here")


# speedup vs baseline: 2.3363x; 2.3363x over previous
import jax
import jax.numpy as jnp
from jax.experimental import pallas as pl


_BLOCK_ROWS = 512


def _add_body(x_ref, p_ref, o_ref):
    o_ref[...] = x_ref[...] + p_ref[...]


def kernel(x, position_matrix):
    n, d = x.shape
    grid = (n // _BLOCK_ROWS,)
    spec = pl.BlockSpec((_BLOCK_ROWS, d), lambda i: (i, 0))
    return pl.pallas_call(
        _add_body,
        grid=grid,
        in_specs=[spec, spec],
        out_specs=spec,
        out_shape=jax.ShapeDtypeStruct((n, d), x.dtype),
    )(x, position_matrix)
